# Initial kernel scaffold; baseline (speedup 1.0000x reference)
import jax
import jax.numpy as jnp
from jax.experimental import pallas as pl


def kernel(qk, v, random_rotations):
    raise NotImplementedError("write your pallas kernel here")


# speedup vs baseline: 2.3292x; 2.3292x over previous
import functools

import jax
import jax.numpy as jnp
from jax import lax
from jax.experimental import pallas as pl
from jax.experimental.pallas import tpu as pltpu

B = 2
S = 4096
D = 64
H = 8
NB = 64
BKT = 64
NBINS = H * (S // BKT)
CHUNK = 512
BINS_PER = 32
ROWS = BINS_PER * BKT
CCH = 512


def _hash_sort_kernel(qk_ref, rot_ref, bkt_ref, fg_ref):
    b = pl.program_id(0)
    h = pl.program_id(1)
    qk = qk_ref[0]
    rot = rot_ref[0]
    r = jnp.dot(qk, rot, preferred_element_type=jnp.float32)
    scores = jnp.concatenate([r, -r], axis=1)
    m = jnp.max(scores, axis=1, keepdims=True)
    lane = lax.broadcasted_iota(jnp.int32, (S, NB), 1)
    bkt = jnp.min(jnp.where(scores == m, lane, NB), axis=1, keepdims=True)
    onehot = (bkt == lane).astype(jnp.float32)
    counts = jnp.sum(onehot, axis=0, keepdims=True)
    i0 = lax.broadcasted_iota(jnp.int32, (NB, NB), 0)
    i1 = lax.broadcasted_iota(jnp.int32, (NB, NB), 1)
    upper = (i0 < i1).astype(jnp.float32)
    offs = jnp.dot(counts, upper, preferred_element_type=jnp.float32)
    c0 = lax.broadcasted_iota(jnp.int32, (CHUNK, CHUNK), 0)
    c1 = lax.broadcasted_iota(jnp.int32, (CHUNK, CHUNK), 1)
    lower = (c0 >= c1).astype(jnp.bfloat16)
    flatbase = (b * H + h) * S
    carry = jnp.zeros((1, NB), jnp.float32)
    for ci in range(S // CHUNK):
        seg = onehot[ci * CHUNK:(ci + 1) * CHUNK]
        incl = jnp.dot(lower, seg.astype(jnp.bfloat16),
                       preferred_element_type=jnp.float32)
        base = offs + carry - 1.0
        pos = jnp.sum(seg * (incl + base), axis=1, keepdims=True)
        fg_ref[0, 0, ci * CHUNK:(ci + 1) * CHUNK, :] = (
            pos.astype(jnp.int32) + flatbase)
        carry = carry + incl[CHUNK - 1:CHUNK, :]
    bkt_ref[0, 0, :, :] = bkt + h * NB


def _hash_sort(qk, rotw):
    grid = (B, H)
    return pl.pallas_call(
        _hash_sort_kernel,
        grid=grid,
        in_specs=[
            pl.BlockSpec((1, S, D), lambda b, h: (b, 0, 0)),
            pl.BlockSpec((1, D, NB // 2), lambda b, h: (h, 0, 0)),
        ],
        out_specs=[
            pl.BlockSpec((1, 1, S, 1), lambda b, h: (b, h, 0, 0)),
            pl.BlockSpec((1, 1, S, 1), lambda b, h: (b, h, 0, 0)),
        ],
        out_shape=[
            jax.ShapeDtypeStruct((B, H, S, 1), jnp.int32),
            jax.ShapeDtypeStruct((B, H, S, 1), jnp.int32),
        ],
    )(qk, rotw)


def _attn_kernel(sqk_ref, sv_ref, str_ref, stc_ref,
                 qk_last_ref, v_last_ref, st_last_ref,
                 so_ref, lse_ref, kprev, vprev, stprev):
    c = pl.program_id(1)

    @pl.when(c == 0)
    def _():
        kprev[...] = qk_last_ref[0]
        vprev[...] = v_last_ref[0]
        stprev[...] = st_last_ref[0]

    for i in range(BINS_PER):
        q = sqk_ref[0, i * BKT:(i + 1) * BKT, :]
        vcur = sv_ref[0, i * BKT:(i + 1) * BKT, :]
        if i == 0:
            kp = kprev[...]
            vp = vprev[...]
            sp = stprev[...]
        else:
            kp = sqk_ref[0, (i - 1) * BKT:i * BKT, :]
            vp = sv_ref[0, (i - 1) * BKT:i * BKT, :]
            sp = str_ref[0, i - 1:i, :]
        kk = jnp.concatenate([q, kp], axis=0)
        vv = jnp.concatenate([vcur, vp], axis=0)
        norm = jnp.sqrt(jnp.sum(kk * kk, axis=1, keepdims=True))
        bk = kk / (norm + 1e-6)
        dots = lax.dot_general(q, bk, (((1,), (1,)), ((), ())),
                               preferred_element_type=jnp.float32)
        dots = dots * (D ** -0.5)
        stq = stc_ref[0, i, :, :]
        stk = jnp.concatenate([str_ref[0, i:i + 1, :], sp], axis=1)
        dots = jnp.where(stq == stk, -100000.0, dots)
        mx = jnp.max(dots, axis=1, keepdims=True)
        p = jnp.exp(dots - mx)
        sm = jnp.sum(p, axis=1, keepdims=True)
        lse = mx + jnp.log(sm)
        w = p / sm
        bo = jnp.dot(w, vv, preferred_element_type=jnp.float32)
        so_ref[0, i * BKT:(i + 1) * BKT, :] = bo
        lse_ref[0, i * BKT:(i + 1) * BKT, :] = lse

    kprev[...] = sqk_ref[0, (BINS_PER - 1) * BKT:BINS_PER * BKT, :]
    vprev[...] = sv_ref[0, (BINS_PER - 1) * BKT:BINS_PER * BKT, :]
    stprev[...] = str_ref[0, BINS_PER - 1:BINS_PER, :]


def _attention(sqk, sv, str_, stc, qk_last, v_last, st_last):
    grid = (B, (H * S) // ROWS)
    return pl.pallas_call(
        _attn_kernel,
        grid=grid,
        in_specs=[
            pl.BlockSpec((1, ROWS, D), lambda b, c: (b, c, 0)),
            pl.BlockSpec((1, ROWS, D), lambda b, c: (b, c, 0)),
            pl.BlockSpec((1, BINS_PER, BKT), lambda b, c: (b, c, 0)),
            pl.BlockSpec((1, BINS_PER, BKT, 1), lambda b, c: (b, c, 0, 0)),
            pl.BlockSpec((1, BKT, D), lambda b, c: (b, 0, 0)),
            pl.BlockSpec((1, BKT, D), lambda b, c: (b, 0, 0)),
            pl.BlockSpec((1, 1, BKT), lambda b, c: (b, 0, 0)),
        ],
        out_specs=[
            pl.BlockSpec((1, ROWS, D), lambda b, c: (b, c, 0)),
            pl.BlockSpec((1, ROWS, 1), lambda b, c: (b, c, 0)),
        ],
        out_shape=[
            jax.ShapeDtypeStruct((B, H * S, D), jnp.float32),
            jax.ShapeDtypeStruct((B, H * S, 1), jnp.float32),
        ],
        scratch_shapes=[
            pltpu.VMEM((BKT, D), jnp.float32),
            pltpu.VMEM((BKT, D), jnp.float32),
            pltpu.VMEM((1, BKT), jnp.int32),
        ],
    )(sqk, sv, str_, stc, qk_last, v_last, st_last)


def _combine_kernel(o_ref, lg_ref, out_ref):
    l = lg_ref[0]
    m = jnp.max(l, axis=0, keepdims=True)
    p = jnp.exp(l - m)
    s = jnp.sum(p, axis=0, keepdims=True)
    w = p / s
    out_ref[0] = jnp.sum(o_ref[0] * w, axis=0)


def _combine(o, lg):
    grid = (B, S // CCH)
    return pl.pallas_call(
        _combine_kernel,
        grid=grid,
        in_specs=[
            pl.BlockSpec((1, H, CCH, D), lambda b, c: (b, 0, c, 0)),
            pl.BlockSpec((1, H, CCH, 1), lambda b, c: (b, 0, c, 0)),
        ],
        out_specs=pl.BlockSpec((1, CCH, D), lambda b, c: (b, c, 0)),
        out_shape=jax.ShapeDtypeStruct((B, S, D), jnp.float32),
    )(o, lg)


def kernel(qk, v, random_rotations):
    rotw = jnp.transpose(random_rotations[0], (1, 0, 2)).reshape(H, D, NB // 2)
    bkt4, fg4 = _hash_sort(qk, rotw)
    buckets = bkt4.reshape(B, H * S)
    fg = fg4.reshape(B * H * S)

    src = jnp.broadcast_to(qk[:, None], (B, H, S, D)).reshape(B * H * S, D)
    vsrc = jnp.broadcast_to(v[:, None], (B, H, S, D)).reshape(B * H * S, D)
    tvals = jnp.broadcast_to(
        jnp.arange(S, dtype=jnp.int32)[None, None, :], (B, H, S)).reshape(-1)
    sqk = jnp.zeros((B * H * S, D), jnp.float32).at[fg].set(src)
    sv = jnp.zeros((B * H * S, D), jnp.float32).at[fg].set(vsrc)
    st = jnp.zeros((B * H * S,), jnp.int32).at[fg].set(tvals)

    sqk = sqk.reshape(B, H * S, D)
    sv = sv.reshape(B, H * S, D)
    str_ = st.reshape(B, NBINS, BKT)
    stc = str_.reshape(B, NBINS, BKT, 1)
    qk_last = sqk[:, -BKT:, :]
    v_last = sv[:, -BKT:, :]
    st_last = str_[:, -1:, :]

    so, lse = _attention(sqk, sv, str_, stc, qk_last, v_last, st_last)

    o = so.reshape(B * H * S, D)[fg].reshape(B, H, S, D)
    lg = lse.reshape(B * H * S)[fg].reshape(B, H, S, 1)

    out = _combine(o, lg)
    return out, buckets


# speedup vs baseline: 3.2337x; 1.3883x over previous
import functools

import jax
import jax.numpy as jnp
from jax import lax
from jax.experimental import pallas as pl
from jax.experimental.pallas import tpu as pltpu
from jax.experimental.pallas import tpu_sc as plsc

B = 2
S = 4096
D = 64
H = 8
NB = 64
BKT = 64
NBINS = H * (S // BKT)
CHUNK = 512
BINS_PER = 32
ROWS = BINS_PER * BKT
CCH = 512


def _hash_sort_kernel(qk_ref, rot_ref, bkt_ref, fg_ref):
    b = pl.program_id(0)
    h = pl.program_id(1)
    qk = qk_ref[0]
    rot = rot_ref[0]
    r = jnp.dot(qk, rot, preferred_element_type=jnp.float32)
    scores = jnp.concatenate([r, -r], axis=1)
    m = jnp.max(scores, axis=1, keepdims=True)
    lane = lax.broadcasted_iota(jnp.int32, (S, NB), 1)
    bkt = jnp.min(jnp.where(scores == m, lane, NB), axis=1, keepdims=True)
    onehot = (bkt == lane).astype(jnp.float32)
    counts = jnp.sum(onehot, axis=0, keepdims=True)
    i0 = lax.broadcasted_iota(jnp.int32, (NB, NB), 0)
    i1 = lax.broadcasted_iota(jnp.int32, (NB, NB), 1)
    upper = (i0 < i1).astype(jnp.float32)
    offs = jnp.dot(counts, upper, preferred_element_type=jnp.float32)
    c0 = lax.broadcasted_iota(jnp.int32, (CHUNK, CHUNK), 0)
    c1 = lax.broadcasted_iota(jnp.int32, (CHUNK, CHUNK), 1)
    lower = (c0 >= c1).astype(jnp.bfloat16)
    flatbase = (b * H + h) * S
    carry = jnp.zeros((1, NB), jnp.float32)
    for ci in range(S // CHUNK):
        seg = onehot[ci * CHUNK:(ci + 1) * CHUNK]
        incl = jnp.dot(lower, seg.astype(jnp.bfloat16),
                       preferred_element_type=jnp.float32)
        base = offs + carry - 1.0
        pos = jnp.sum(seg * (incl + base), axis=1, keepdims=True)
        fg_ref[0, 0, ci * CHUNK:(ci + 1) * CHUNK, :] = (
            pos.astype(jnp.int32) + flatbase)
        carry = carry + incl[CHUNK - 1:CHUNK, :]
    bkt_ref[0, 0, :, :] = bkt + h * NB


def _hash_sort(qk, rotw):
    grid = (B, H)
    return pl.pallas_call(
        _hash_sort_kernel,
        grid=grid,
        in_specs=[
            pl.BlockSpec((1, S, D), lambda b, h: (b, 0, 0)),
            pl.BlockSpec((1, D, NB // 2), lambda b, h: (h, 0, 0)),
        ],
        out_specs=[
            pl.BlockSpec((1, 1, S, 1), lambda b, h: (b, h, 0, 0)),
            pl.BlockSpec((1, 1, S, 1), lambda b, h: (b, h, 0, 0)),
        ],
        out_shape=[
            jax.ShapeDtypeStruct((B, H, S, 1), jnp.int32),
            jax.ShapeDtypeStruct((B, H, S, 1), jnp.int32),
        ],
    )(qk, rotw)


def _attn_kernel(sqk_ref, sv_ref, str_ref, stc_ref,
                 qk_last_ref, v_last_ref, st_last_ref,
                 so_ref, lse_ref, kprev, vprev, stprev):
    c = pl.program_id(1)

    @pl.when(c == 0)
    def _():
        kprev[...] = qk_last_ref[0]
        vprev[...] = v_last_ref[0]
        stprev[...] = st_last_ref[0]

    for i in range(BINS_PER):
        q = sqk_ref[0, i * BKT:(i + 1) * BKT, :]
        vcur = sv_ref[0, i * BKT:(i + 1) * BKT, :]
        if i == 0:
            kp = kprev[...]
            vp = vprev[...]
            sp = stprev[...]
        else:
            kp = sqk_ref[0, (i - 1) * BKT:i * BKT, :]
            vp = sv_ref[0, (i - 1) * BKT:i * BKT, :]
            sp = str_ref[0, i - 1:i, :]
        kk = jnp.concatenate([q, kp], axis=0)
        vv = jnp.concatenate([vcur, vp], axis=0)
        norm = jnp.sqrt(jnp.sum(kk * kk, axis=1, keepdims=True))
        bk = kk / (norm + 1e-6)
        dots = lax.dot_general(q, bk, (((1,), (1,)), ((), ())),
                               preferred_element_type=jnp.float32)
        dots = dots * (D ** -0.5)
        stq = stc_ref[0, i, :, :]
        stk = jnp.concatenate([str_ref[0, i:i + 1, :], sp], axis=1)
        dots = jnp.where(stq == stk, -100000.0, dots)
        mx = jnp.max(dots, axis=1, keepdims=True)
        p = jnp.exp(dots - mx)
        sm = jnp.sum(p, axis=1, keepdims=True)
        lse = mx + jnp.log(sm)
        w = p / sm
        bo = jnp.dot(w, vv, preferred_element_type=jnp.float32)
        so_ref[0, i * BKT:(i + 1) * BKT, :] = bo
        lse_ref[0, i * BKT:(i + 1) * BKT, :] = lse

    kprev[...] = sqk_ref[0, (BINS_PER - 1) * BKT:BINS_PER * BKT, :]
    vprev[...] = sv_ref[0, (BINS_PER - 1) * BKT:BINS_PER * BKT, :]
    stprev[...] = str_ref[0, BINS_PER - 1:BINS_PER, :]


def _attention(sqk, sv, str_, stc, qk_last, v_last, st_last):
    grid = (B, (H * S) // ROWS)
    return pl.pallas_call(
        _attn_kernel,
        grid=grid,
        in_specs=[
            pl.BlockSpec((1, ROWS, D), lambda b, c: (b, c, 0)),
            pl.BlockSpec((1, ROWS, D), lambda b, c: (b, c, 0)),
            pl.BlockSpec((1, BINS_PER, BKT), lambda b, c: (b, c, 0)),
            pl.BlockSpec((1, BINS_PER, BKT, 1), lambda b, c: (b, c, 0, 0)),
            pl.BlockSpec((1, BKT, D), lambda b, c: (b, 0, 0)),
            pl.BlockSpec((1, BKT, D), lambda b, c: (b, 0, 0)),
            pl.BlockSpec((1, 1, BKT), lambda b, c: (b, 0, 0)),
        ],
        out_specs=[
            pl.BlockSpec((1, ROWS, D), lambda b, c: (b, c, 0)),
            pl.BlockSpec((1, ROWS, 1), lambda b, c: (b, c, 0)),
        ],
        out_shape=[
            jax.ShapeDtypeStruct((B, H * S, D), jnp.float32),
            jax.ShapeDtypeStruct((B, H * S, 1), jnp.float32),
        ],
        scratch_shapes=[
            pltpu.VMEM((BKT, D), jnp.float32),
            pltpu.VMEM((BKT, D), jnp.float32),
            pltpu.VMEM((1, BKT), jnp.int32),
        ],
    )(sqk, sv, str_, stc, qk_last, v_last, st_last)


def _combine_kernel(o_ref, lg_ref, out_ref):
    l = lg_ref[0]
    m = jnp.max(l, axis=0, keepdims=True)
    p = jnp.exp(l - m)
    s = jnp.sum(p, axis=0, keepdims=True)
    w = p / s
    out_ref[0] = jnp.sum(o_ref[0] * w, axis=0)


def _combine(o, lg):
    grid = (B, S // CCH)
    return pl.pallas_call(
        _combine_kernel,
        grid=grid,
        in_specs=[
            pl.BlockSpec((1, H, CCH, D), lambda b, c: (b, 0, c, 0)),
            pl.BlockSpec((1, H, CCH, 1), lambda b, c: (b, 0, c, 0)),
        ],
        out_specs=pl.BlockSpec((1, CCH, D), lambda b, c: (b, c, 0)),
        out_shape=jax.ShapeDtypeStruct((B, S, D), jnp.float32),
    )(o, lg)


NW = 32
TPT = (B * H * S) // NW
K = 128


def _sc_wid():
    return lax.axis_index("s") * 2 + lax.axis_index("c")


def _sc_scatter_body(qk_hbm, v_hbm, fg_hbm, tarr_hbm,
                     sqk_hbm, sv_hbm, st_hbm,
                     idx_v, rows_v, vrows_v, tv_v, sem):
    wid = _sc_wid()
    pair = wid // 2
    half = wid % 2
    b = pair // H
    fgbase = pair * S + half * (S // 2)
    qkbase = b * S + half * (S // 2)

    def chunk(ck, _):
        t0 = ck * K
        pltpu.sync_copy(fg_hbm.at[pl.ds(fgbase + t0, K)], idx_v)
        pltpu.sync_copy(qk_hbm.at[pl.ds(qkbase + t0, K)], rows_v)
        pltpu.sync_copy(v_hbm.at[pl.ds(qkbase + t0, K)], vrows_v)
        pltpu.sync_copy(tarr_hbm.at[pl.ds(half * (S // 2) + t0, K)], tv_v)
        d1 = pltpu.async_copy(rows_v, sqk_hbm.at[idx_v], sem)
        d2 = pltpu.async_copy(vrows_v, sv_hbm.at[idx_v], sem)
        d3 = pltpu.async_copy(tv_v, st_hbm.at[idx_v], sem)
        d1.wait()
        d2.wait()
        d3.wait()
        return 0

    lax.fori_loop(0, TPT // K, chunk, 0)


def _sc_scatter(qk_flat, v_flat, fg, tarr):
    fn = pl.kernel(
        _sc_scatter_body,
        out_type=[
            jax.ShapeDtypeStruct((B * H * S, D), jnp.float32),
            jax.ShapeDtypeStruct((B * H * S, D), jnp.float32),
            jax.ShapeDtypeStruct((B * H * S,), jnp.int32),
        ],
        mesh=plsc.VectorSubcoreMesh(core_axis_name="c", subcore_axis_name="s", num_cores=2, num_subcores=16),
        compiler_params=pltpu.CompilerParams(use_tc_tiling_on_sc=False),
        scratch_types=[
            pltpu.VMEM((K,), jnp.int32),
            pltpu.VMEM((K, D), jnp.float32),
            pltpu.VMEM((K, D), jnp.float32),
            pltpu.VMEM((K,), jnp.int32),
            pltpu.SemaphoreType.DMA,
        ],
    )
    return fn(qk_flat, v_flat, fg, tarr)


def _sc_gather_body(so_hbm, lse_hbm, fg_hbm,
                    o_hbm, lg_hbm,
                    idx_v, rows_v, l_v, sem):
    wid = _sc_wid()
    pair = wid // 2
    half = wid % 2
    base = pair * S + half * (S // 2)

    def chunk(ck, _):
        t0 = ck * K
        pltpu.sync_copy(fg_hbm.at[pl.ds(base + t0, K)], idx_v)
        d1 = pltpu.async_copy(so_hbm.at[idx_v], rows_v, sem)
        d2 = pltpu.async_copy(lse_hbm.at[idx_v], l_v, sem)
        d1.wait()
        d2.wait()
        pltpu.sync_copy(rows_v, o_hbm.at[pl.ds(base + t0, K)])
        pltpu.sync_copy(l_v, lg_hbm.at[pl.ds(base + t0, K)])
        return 0

    lax.fori_loop(0, TPT // K, chunk, 0)


def _sc_gather(so_flat, lse_flat, fg):
    fn = pl.kernel(
        _sc_gather_body,
        out_type=[
            jax.ShapeDtypeStruct((B * H * S, D), jnp.float32),
            jax.ShapeDtypeStruct((B * H * S,), jnp.float32),
        ],
        mesh=plsc.VectorSubcoreMesh(core_axis_name="c", subcore_axis_name="s", num_cores=2, num_subcores=16),
        compiler_params=pltpu.CompilerParams(use_tc_tiling_on_sc=False),
        scratch_types=[
            pltpu.VMEM((K,), jnp.int32),
            pltpu.VMEM((K, D), jnp.float32),
            pltpu.VMEM((K,), jnp.float32),
            pltpu.SemaphoreType.DMA,
        ],
    )
    return fn(so_flat, lse_flat, fg)


def kernel(qk, v, random_rotations):
    rotw = jnp.transpose(random_rotations[0], (1, 0, 2)).reshape(H, D, NB // 2)
    bkt4, fg4 = _hash_sort(qk, rotw)
    buckets = bkt4.reshape(B, H * S)
    fg = fg4.reshape(B * H * S)

    tarr = jnp.arange(S, dtype=jnp.int32)
    sqk, sv, st = _sc_scatter(qk.reshape(B * S, D), v.reshape(B * S, D),
                              fg, tarr)

    sqk = sqk.reshape(B, H * S, D)
    sv = sv.reshape(B, H * S, D)
    str_ = st.reshape(B, NBINS, BKT)
    stc = str_.reshape(B, NBINS, BKT, 1)
    qk_last = sqk[:, -BKT:, :]
    v_last = sv[:, -BKT:, :]
    st_last = str_[:, -1:, :]

    so, lse = _attention(sqk, sv, str_, stc, qk_last, v_last, st_last)

    o_flat, lg_flat = _sc_gather(so.reshape(B * H * S, D),
                                 lse.reshape(B * H * S), fg)
    o = o_flat.reshape(B, H, S, D)
    lg = lg_flat.reshape(B, H, S, 1)

    out = _combine(o, lg)
    return out, buckets


# speedup vs baseline: 3.2920x; 1.0180x over previous
import jax
import jax.numpy as jnp
from jax import lax
from jax.experimental import pallas as pl
from jax.experimental.pallas import tpu as pltpu
from jax.experimental.pallas import tpu_sc as plsc

B = 2
S = 4096
D = 64
H = 8
NB = 64
BKT = 64
CHUNK = 512
BINS_PER = 32
ROWS = BINS_PER * BKT
NCH = (H * S) // ROWS
CCH = 512
P = 2 * D

NW = 32
TPT = (B * H * S) // NW
K = 128
NCK = TPT // K

_SC_MESH = dict(core_axis_name="c", subcore_axis_name="s",
                num_cores=2, num_subcores=16)


def _hash_sort_kernel(qk_ref, rot_ref, bkt_ref, fg_ref):
    b = pl.program_id(0)
    h = pl.program_id(1)
    qk = qk_ref[0]
    rot = rot_ref[0]
    r = jnp.dot(qk, rot, preferred_element_type=jnp.float32)
    scores = jnp.concatenate([r, -r], axis=1)
    m = jnp.max(scores, axis=1, keepdims=True)
    lane = lax.broadcasted_iota(jnp.int32, (S, NB), 1)
    bkt = jnp.min(jnp.where(scores == m, lane, NB), axis=1, keepdims=True)
    onehot = (bkt == lane).astype(jnp.float32)
    bktf = bkt.astype(jnp.float32)
    counts = jnp.sum(onehot, axis=0, keepdims=True)
    i0 = lax.broadcasted_iota(jnp.int32, (NB, NB), 0)
    i1 = lax.broadcasted_iota(jnp.int32, (NB, NB), 1)
    upper = (i0 < i1).astype(jnp.float32)
    offs = jnp.dot(counts, upper, preferred_element_type=jnp.float32,
                   precision=lax.Precision.HIGHEST)
    c0 = lax.broadcasted_iota(jnp.int32, (CHUNK, CHUNK), 0)
    c1 = lax.broadcasted_iota(jnp.int32, (CHUNK, CHUNK), 1)
    lower = (c0 >= c1).astype(jnp.float32)
    ident = (c0 == c1).astype(jnp.float32)
    flatbase = ((b * H + h) * S).astype(jnp.float32)
    hoff = (h * NB).astype(jnp.float32)
    carry = jnp.zeros((1, NB), jnp.float32)
    for ci in range(S // CHUNK):
        sl = slice(ci * CHUNK, (ci + 1) * CHUNK)
        seg = onehot[sl]
        incl = jnp.dot(lower, seg, preferred_element_type=jnp.float32)
        base = offs + carry - 1.0
        pos = jnp.sum(seg * (incl + base), axis=1, keepdims=True)
        posr = lax.dot_general(pos, ident, (((0,), (0,)), ((), ())),
                               preferred_element_type=jnp.float32,
                               precision=lax.Precision.HIGHEST)
        bktr = lax.dot_general(bktf[sl], ident, (((0,), (0,)), ((), ())),
                               preferred_element_type=jnp.float32,
                               precision=lax.Precision.HIGHEST)
        fg_ref[0, 0, 0:1, sl] = (posr + flatbase).astype(jnp.int32)
        bkt_ref[0, 0, 0:1, sl] = (bktr + hoff).astype(jnp.int32)
        carry = carry + incl[CHUNK - 1:CHUNK, :]


def _hash_sort(qk, rotw):
    return pl.pallas_call(
        _hash_sort_kernel,
        grid=(B, H),
        in_specs=[
            pl.BlockSpec((1, S, D), lambda b, h: (b, 0, 0)),
            pl.BlockSpec((1, D, NB // 2), lambda b, h: (h, 0, 0)),
        ],
        out_specs=[
            pl.BlockSpec((1, 1, 1, S), lambda b, h: (b, h, 0, 0)),
            pl.BlockSpec((1, 1, 1, S), lambda b, h: (b, h, 0, 0)),
        ],
        out_shape=[
            jax.ShapeDtypeStruct((B, H, 1, S), jnp.int32),
            jax.ShapeDtypeStruct((B, H, 1, S), jnp.int32),
        ],
    )(qk, rotw)


def _sc_wid():
    return lax.axis_index("s") * 2 + lax.axis_index("c")


def _sc_scatter_body(qkv_hbm, fg_hbm, tarr_hbm,
                     sqkv_hbm, st_hbm,
                     idx0, idx1, rows0, rows1, tv0, tv1, sem_in, sem_out):
    wid = _sc_wid()
    pair = wid // 2
    half = wid % 2
    b = pair // H
    fgbase = pair * S + half * (S // 2)
    qkbase = b * S + half * (S // 2)
    tbase = half * (S // 2)
    idx = (idx0, idx1)
    rows = (rows0, rows1)
    tv = (tv0, tv1)
    pend = {}
    for ck in range(NCK):
        s = ck % 2
        if ck >= 2:
            pend[s][0].wait()
            pend[s][1].wait()
        t0 = ck * K
        l1 = pltpu.async_copy(fg_hbm.at[pl.ds(fgbase + t0, K)], idx[s], sem_in)
        l2 = pltpu.async_copy(qkv_hbm.at[pl.ds(qkbase + t0, K)], rows[s], sem_in)
        l3 = pltpu.async_copy(tarr_hbm.at[pl.ds(tbase + t0, K)], tv[s], sem_in)
        l1.wait()
        l2.wait()
        l3.wait()
        d1 = pltpu.async_copy(rows[s], sqkv_hbm.at[idx[s]], sem_out)
        d2 = pltpu.async_copy(tv[s], st_hbm.at[idx[s]], sem_out)
        pend[s] = (d1, d2)
    for s in (0, 1):
        pend[s][0].wait()
        pend[s][1].wait()


def _sc_scatter(qkv, fg, tarr):
    fn = pl.kernel(
        _sc_scatter_body,
        out_type=[
            jax.ShapeDtypeStruct((B * H * S, P), jnp.float32),
            jax.ShapeDtypeStruct((B * H * S,), jnp.float32),
        ],
        mesh=plsc.VectorSubcoreMesh(**_SC_MESH),
        compiler_params=pltpu.CompilerParams(use_tc_tiling_on_sc=False),
        scratch_types=[
            pltpu.VMEM((K,), jnp.int32),
            pltpu.VMEM((K,), jnp.int32),
            pltpu.VMEM((K, P), jnp.float32),
            pltpu.VMEM((K, P), jnp.float32),
            pltpu.VMEM((K,), jnp.float32),
            pltpu.VMEM((K,), jnp.float32),
            pltpu.SemaphoreType.DMA,
            pltpu.SemaphoreType.DMA,
        ],
    )
    return fn(qkv, fg, tarr)


def _sc_gather_body(sog_hbm, fg_hbm, og_hbm,
                    idx0, idx1, rows0, rows1, sem_in, sem_out):
    wid = _sc_wid()
    base = (wid // 2) * S + (wid % 2) * (S // 2)
    idx = (idx0, idx1)
    rows = (rows0, rows1)
    pend = {}
    for ck in range(NCK):
        s = ck % 2
        if ck >= 2:
            pend[s].wait()
        t0 = ck * K
        l1 = pltpu.async_copy(fg_hbm.at[pl.ds(base + t0, K)], idx[s], sem_in)
        l1.wait()
        g = pltpu.async_copy(sog_hbm.at[idx[s]], rows[s], sem_in)
        g.wait()
        pend[s] = pltpu.async_copy(rows[s], og_hbm.at[pl.ds(base + t0, K)],
                                   sem_out)
    for s in (0, 1):
        pend[s].wait()


def _sc_gather(sog, fg):
    fn = pl.kernel(
        _sc_gather_body,
        out_type=jax.ShapeDtypeStruct((B * H * S, P), jnp.float32),
        mesh=plsc.VectorSubcoreMesh(**_SC_MESH),
        compiler_params=pltpu.CompilerParams(use_tc_tiling_on_sc=False),
        scratch_types=[
            pltpu.VMEM((K,), jnp.int32),
            pltpu.VMEM((K,), jnp.int32),
            pltpu.VMEM((K, P), jnp.float32),
            pltpu.VMEM((K, P), jnp.float32),
            pltpu.SemaphoreType.DMA,
            pltpu.SemaphoreType.DMA,
        ],
    )
    return fn(sog, fg)


def _attn_kernel(kv_ref, st_ref, kv_last_ref, st_last_ref,
                 sog_ref, kvprev, stprev):
    c = pl.program_id(1)

    @pl.when(c == 0)
    def _():
        kvprev[...] = kv_last_ref[0]
        stprev[...] = st_last_ref[0]

    e0 = lax.broadcasted_iota(jnp.int32, (BKT, BKT), 0)
    e1 = lax.broadcasted_iota(jnp.int32, (BKT, BKT), 1)
    ident = (e0 == e1).astype(jnp.float32)

    for i in range(BINS_PER):
        cur = kv_ref[i * BKT:(i + 1) * BKT, :]
        prev = kvprev[...] if i == 0 else kv_ref[(i - 1) * BKT:i * BKT, :]
        sp = stprev[...] if i == 0 else st_ref[0, 0:1, (i - 1) * BKT:i * BKT]
        q = cur[:, 0:D]
        kk = jnp.concatenate([q, prev[:, 0:D]], axis=0)
        vv = jnp.concatenate([cur[:, D:P], prev[:, D:P]], axis=0)
        norm = jnp.sqrt(jnp.sum(kk * kk, axis=1, keepdims=True))
        bk = kk / (norm + 1e-6)
        dots = lax.dot_general(q, bk, (((1,), (1,)), ((), ())),
                               preferred_element_type=jnp.float32)
        dots = dots * (D ** -0.5)
        strow = st_ref[0, 0:1, i * BKT:(i + 1) * BKT]
        stq = lax.dot_general(ident, strow, (((1,), (1,)), ((), ())),
                              preferred_element_type=jnp.float32,
                              precision=lax.Precision.HIGHEST)
        stk = jnp.concatenate([strow, sp], axis=1)
        dots = jnp.where(stq == stk, -100000.0, dots)
        mx = jnp.max(dots, axis=1, keepdims=True)
        p = jnp.exp(dots - mx)
        sm = jnp.sum(p, axis=1, keepdims=True)
        lse = mx + jnp.log(sm)
        w = p / sm
        bo = jnp.dot(w, vv, preferred_element_type=jnp.float32)
        lse_b = jnp.broadcast_to(lse, (BKT, D))
        sog_ref[i * BKT:(i + 1) * BKT, :] = jnp.concatenate([bo, lse_b], axis=1)

    kvprev[...] = kv_ref[(BINS_PER - 1) * BKT:ROWS, :]
    stprev[...] = st_ref[0, 0:1, (BINS_PER - 1) * BKT:ROWS]


def _attention(sqkv, str3, kv_last, st_last):
    return pl.pallas_call(
        _attn_kernel,
        grid=(B, NCH),
        in_specs=[
            pl.BlockSpec((ROWS, P), lambda b, c: (b * NCH + c, 0)),
            pl.BlockSpec((1, 1, ROWS), lambda b, c: (b * NCH + c, 0, 0)),
            pl.BlockSpec((1, BKT, P), lambda b, c: (b, 0, 0)),
            pl.BlockSpec((1, 1, BKT), lambda b, c: (b, 0, 0)),
        ],
        out_specs=pl.BlockSpec((ROWS, P), lambda b, c: (b * NCH + c, 0)),
        out_shape=jax.ShapeDtypeStruct((B * H * S, P), jnp.float32),
        scratch_shapes=[
            pltpu.VMEM((BKT, P), jnp.float32),
            pltpu.VMEM((1, BKT), jnp.float32),
        ],
    )(sqkv, str3, kv_last, st_last)


def _combine_kernel(og_ref, out_ref):
    o = og_ref[:, :, 0:D]
    l = jnp.max(og_ref[:, :, D:P], axis=2, keepdims=True)
    m = jnp.max(l, axis=0, keepdims=True)
    p = jnp.exp(l - m)
    s = jnp.sum(p, axis=0, keepdims=True)
    w = p / s
    out_ref[0] = jnp.sum(o * w, axis=0)


def _combine(og3):
    return pl.pallas_call(
        _combine_kernel,
        grid=(B, S // CCH),
        in_specs=[
            pl.BlockSpec((H, CCH, P), lambda b, c: (b, c, 0)),
        ],
        out_specs=pl.BlockSpec((1, CCH, D), lambda b, c: (b, c, 0)),
        out_shape=jax.ShapeDtypeStruct((B, S, D), jnp.float32),
    )(og3)


def kernel(qk, v, random_rotations):
    rotw = jnp.transpose(random_rotations[0], (1, 0, 2)).reshape(H, D, NB // 2)
    bkt4, fg4 = _hash_sort(qk, rotw)
    buckets = bkt4.reshape(B, H * S)
    fg = fg4.reshape(B * H * S)

    qkv = jnp.concatenate([qk, v], axis=-1).reshape(B * S, P)
    tarr = jnp.arange(S, dtype=jnp.float32)
    sqkv, st = _sc_scatter(qkv, fg, tarr)

    str3 = st.reshape(B * NCH, 1, ROWS)
    kv_last = sqkv.reshape(B, H * S, P)[:, -BKT:, :]
    st_last = st.reshape(B, H * S)[:, -BKT:].reshape(B, 1, BKT)

    sog = _attention(sqkv, str3, kv_last, st_last)

    og = _sc_gather(sog, fg)
    out = _combine(og.reshape(B * H, S, P))
    return out, buckets


# speedup vs baseline: 5.7351x; 1.7421x over previous
import jax
import jax.numpy as jnp
from jax import lax
from jax.experimental import pallas as pl
from jax.experimental.pallas import tpu as pltpu
from jax.experimental.pallas import tpu_sc as plsc

B = 2
S = 4096
D = 64
H = 8
NB = 64
BKT = 64
CHUNK = 512
BINS_PER = 32
ROWS = BINS_PER * BKT
NCH = (H * S) // ROWS
CCH = 512
P = 2 * D

NW = 32
TPT = (B * H * S) // NW
K = 128
NCK = TPT // K

_SC_MESH = dict(core_axis_name="c", subcore_axis_name="s",
                num_cores=2, num_subcores=16)


def _hash_sort_kernel(qk_ref, rot_ref, bkt_ref, fg_ref, qt0_ref, ktl_ref):
    b = pl.program_id(0)
    h = pl.program_id(1)
    qk = qk_ref[0]
    rot = rot_ref[0]
    r = jnp.dot(qk, rot, preferred_element_type=jnp.float32)
    scores = jnp.concatenate([r, -r], axis=1)
    m = jnp.max(scores, axis=1, keepdims=True)
    lane = lax.broadcasted_iota(jnp.int32, (S, NB), 1)
    bkt = jnp.min(jnp.where(scores == m, lane, NB), axis=1, keepdims=True)
    onehot = (bkt == lane).astype(jnp.float32)
    bktf = bkt.astype(jnp.float32)
    counts = jnp.sum(onehot, axis=0, keepdims=True)
    i0 = lax.broadcasted_iota(jnp.int32, (NB, NB), 0)
    i1 = lax.broadcasted_iota(jnp.int32, (NB, NB), 1)
    upper = (i0 < i1).astype(jnp.float32)
    offs = jnp.dot(counts, upper, preferred_element_type=jnp.float32,
                   precision=lax.Precision.HIGHEST)
    c0 = lax.broadcasted_iota(jnp.int32, (CHUNK, CHUNK), 0)
    c1 = lax.broadcasted_iota(jnp.int32, (CHUNK, CHUNK), 1)
    lower = (c0 >= c1).astype(jnp.bfloat16)
    ident = (c0 == c1).astype(jnp.float32)
    flatbase = ((b * H + h) * S).astype(jnp.float32)
    hoff = (h * NB).astype(jnp.float32)
    l64 = lax.broadcasted_iota(jnp.int32, (CHUNK, BKT), 1).astype(jnp.float32)
    tcol = lax.broadcasted_iota(jnp.int32, (CHUNK, 1), 0).astype(jnp.float32)
    carry = jnp.zeros((1, NB), jnp.float32)
    qt0a = jnp.zeros((1, BKT), jnp.float32)
    ktla = jnp.zeros((1, BKT), jnp.float32)
    for ci in range(S // CHUNK):
        sl = slice(ci * CHUNK, (ci + 1) * CHUNK)
        seg = onehot[sl]
        incl = jnp.dot(lower, seg.astype(jnp.bfloat16),
                       preferred_element_type=jnp.float32)
        base = offs + carry - 1.0
        pos = jnp.sum(seg * (incl + base), axis=1, keepdims=True)
        posr = lax.dot_general(pos, ident, (((0,), (0,)), ((), ())),
                               preferred_element_type=jnp.float32,
                               precision=lax.Precision.HIGHEST)
        bktr = lax.dot_general(bktf[sl], ident, (((0,), (0,)), ((), ())),
                               preferred_element_type=jnp.float32,
                               precision=lax.Precision.HIGHEST)
        fg_ref[0, 0, 0:1, sl] = (posr + flatbase).astype(jnp.int32)
        bkt_ref[0, 0, 0:1, sl] = (bktr + hoff).astype(jnp.int32)
        carry = carry + incl[CHUNK - 1:CHUNK, :]
        tvals = tcol + float(ci * CHUNK)
        oh0 = (pos == l64).astype(jnp.float32)
        ohl = (pos == l64 + float(S - BKT)).astype(jnp.float32)
        qt0a = qt0a + lax.dot_general(tvals, oh0, (((0,), (0,)), ((), ())),
                                      preferred_element_type=jnp.float32,
                                      precision=lax.Precision.HIGHEST)
        ktla = ktla + lax.dot_general(tvals, ohl, (((0,), (0,)), ((), ())),
                                      preferred_element_type=jnp.float32,
                                      precision=lax.Precision.HIGHEST)
    qt0_ref[0, 0] = qt0a
    ktl_ref[0, 0] = ktla


def _hash_sort(qk, rotw):
    return pl.pallas_call(
        _hash_sort_kernel,
        grid=(B, H),
        in_specs=[
            pl.BlockSpec((1, S, D), lambda b, h: (b, 0, 0)),
            pl.BlockSpec((1, D, NB // 2), lambda b, h: (h, 0, 0)),
        ],
        out_specs=[
            pl.BlockSpec((1, 1, 1, S), lambda b, h: (b, h, 0, 0)),
            pl.BlockSpec((1, 1, 1, S), lambda b, h: (b, h, 0, 0)),
            pl.BlockSpec((1, 1, 1, BKT), lambda b, h: (b, h, 0, 0)),
            pl.BlockSpec((1, 1, 1, BKT), lambda b, h: (b, h, 0, 0)),
        ],
        out_shape=[
            jax.ShapeDtypeStruct((B, H, 1, S), jnp.int32),
            jax.ShapeDtypeStruct((B, H, 1, S), jnp.int32),
            jax.ShapeDtypeStruct((B, H, 1, BKT), jnp.float32),
            jax.ShapeDtypeStruct((B, H, 1, BKT), jnp.float32),
        ],
    )(qk, rotw)


def _sc_wid():
    return lax.axis_index("s") * 2 + lax.axis_index("c")


def _sc_scatter_body(qkv_hbm, fg_hbm, sqkv_hbm,
                     idx0, idx1, rows0, rows1, sem_in, sem_out):
    wid = _sc_wid()
    pair = wid // 2
    half = wid % 2
    b = pair // H
    fgbase = pair * S + half * (S // 2)
    qkbase = b * S + half * (S // 2)
    idx = (idx0, idx1)
    rows = (rows0, rows1)
    pend = {}
    for ck in range(NCK):
        s = ck % 2
        if ck >= 2:
            pend[s].wait()
        t0 = ck * K
        l1 = pltpu.async_copy(fg_hbm.at[pl.ds(fgbase + t0, K)], idx[s], sem_in)
        l2 = pltpu.async_copy(qkv_hbm.at[pl.ds(qkbase + t0, K)], rows[s], sem_in)
        l1.wait()
        l2.wait()
        pend[s] = pltpu.async_copy(rows[s], sqkv_hbm.at[idx[s]], sem_out)
    for s in (0, 1):
        pend[s].wait()


def _sc_scatter(qkv, fg):
    fn = pl.kernel(
        _sc_scatter_body,
        out_type=jax.ShapeDtypeStruct((B * H * S, P), jnp.float32),
        mesh=plsc.VectorSubcoreMesh(**_SC_MESH),
        compiler_params=pltpu.CompilerParams(use_tc_tiling_on_sc=False),
        scratch_types=[
            pltpu.VMEM((K,), jnp.int32),
            pltpu.VMEM((K,), jnp.int32),
            pltpu.VMEM((K, P), jnp.float32),
            pltpu.VMEM((K, P), jnp.float32),
            pltpu.SemaphoreType.DMA,
            pltpu.SemaphoreType.DMA,
        ],
    )
    return fn(qkv, fg)


def _sc_gather_body(sog_hbm, fg_hbm, og_hbm,
                    idx0, idx1, rows0, rows1, sem_in, sem_out):
    wid = _sc_wid()
    base = (wid // 2) * S + (wid % 2) * (S // 2)
    idx = (idx0, idx1)
    rows = (rows0, rows1)
    pend = {}
    for ck in range(NCK):
        s = ck % 2
        if ck >= 2:
            pend[s].wait()
        t0 = ck * K
        l1 = pltpu.async_copy(fg_hbm.at[pl.ds(base + t0, K)], idx[s], sem_in)
        l1.wait()
        g = pltpu.async_copy(sog_hbm.at[idx[s]], rows[s], sem_in)
        g.wait()
        pend[s] = pltpu.async_copy(rows[s], og_hbm.at[pl.ds(base + t0, K)],
                                   sem_out)
    for s in (0, 1):
        pend[s].wait()


def _sc_gather(sog, fg):
    fn = pl.kernel(
        _sc_gather_body,
        out_type=jax.ShapeDtypeStruct((B * H * S, P), jnp.float32),
        mesh=plsc.VectorSubcoreMesh(**_SC_MESH),
        compiler_params=pltpu.CompilerParams(use_tc_tiling_on_sc=False),
        scratch_types=[
            pltpu.VMEM((K,), jnp.int32),
            pltpu.VMEM((K,), jnp.int32),
            pltpu.VMEM((K, P), jnp.float32),
            pltpu.VMEM((K, P), jnp.float32),
            pltpu.SemaphoreType.DMA,
            pltpu.SemaphoreType.DMA,
        ],
    )
    return fn(sog, fg)


def _attn_kernel(kv_ref, kv_last_ref, qt_ref, ktl_ref, sog_ref, kvprev):
    c = pl.program_id(1)

    @pl.when(c == 0)
    def _():
        kvprev[...] = kv_last_ref[0]

    cur = kv_ref[...]
    prev = jnp.concatenate([kvprev[...], cur[:ROWS - BKT]], axis=0)
    kvprev[...] = cur[ROWS - BKT:ROWS]

    cur3 = cur.reshape(BINS_PER, BKT, P)
    prev3 = prev.reshape(BINS_PER, BKT, P)
    q = cur3[:, :, 0:D]
    kk = jnp.concatenate([cur3[:, :, 0:D], prev3[:, :, 0:D]], axis=1)
    vv = jnp.concatenate([cur3[:, :, D:P], prev3[:, :, D:P]], axis=1)
    norm = jnp.sqrt(jnp.sum(kk * kk, axis=2, keepdims=True))
    bk = kk / (norm + 1e-6)
    dots = lax.dot_general(q, bk, (((2,), (2,)), ((0,), (0,))),
                           preferred_element_type=jnp.float32)
    dots = dots * (D ** -0.5)

    i1 = lax.broadcasted_iota(jnp.int32, (BINS_PER, BKT, 2 * BKT), 1)
    i2 = lax.broadcasted_iota(jnp.int32, (BINS_PER, BKT, 2 * BKT), 2)
    mask = i1 == i2
    e0 = lax.broadcasted_iota(jnp.int32, (BKT, BKT), 0)
    e1 = lax.broadcasted_iota(jnp.int32, (BKT, BKT), 1)
    ident = (e0 == e1).astype(jnp.float32)
    qt = qt_ref[0, 0]
    ktl = ktl_ref[0, 0]
    qtT = lax.dot_general(ident, qt, (((1,), (1,)), ((), ())),
                          preferred_element_type=jnp.float32,
                          precision=lax.Precision.HIGHEST)
    ktlp = jnp.concatenate(
        [jnp.full((1, BKT), -1.0, jnp.float32), ktl], axis=1)
    lb = qtT == ktlp
    b0 = lax.broadcasted_iota(jnp.int32, (BINS_PER, BKT, 2 * BKT), 0) == 0
    mask = mask | (b0 & lb[None] & ((c % 2) == 0))
    dots = jnp.where(mask, -100000.0, dots)

    mx = jnp.max(dots, axis=2, keepdims=True)
    p = jnp.exp(dots - mx)
    sm = jnp.sum(p, axis=2, keepdims=True)
    lse = mx + jnp.log(sm)
    w = p / sm
    bo = lax.dot_general(w, vv, (((2,), (1,)), ((0,), (0,))),
                         preferred_element_type=jnp.float32)
    lse_b = jnp.broadcast_to(lse, (BINS_PER, BKT, D))
    sog_ref[...] = jnp.concatenate([bo, lse_b], axis=2).reshape(ROWS, P)


def _attention(sqkv, kv_last, qt0, ktl):
    return pl.pallas_call(
        _attn_kernel,
        grid=(B, NCH),
        in_specs=[
            pl.BlockSpec((ROWS, P), lambda b, c: (b * NCH + c, 0)),
            pl.BlockSpec((1, BKT, P), lambda b, c: (b, 0, 0)),
            pl.BlockSpec((1, 1, 1, BKT), lambda b, c: (b, c // 2, 0, 0)),
            pl.BlockSpec((1, 1, 1, BKT),
                         lambda b, c: (b, (c // 2 + H - 1) % H, 0, 0)),
        ],
        out_specs=pl.BlockSpec((ROWS, P), lambda b, c: (b * NCH + c, 0)),
        out_shape=jax.ShapeDtypeStruct((B * H * S, P), jnp.float32),
        scratch_shapes=[
            pltpu.VMEM((BKT, P), jnp.float32),
        ],
    )(sqkv, kv_last, qt0, ktl)


def _combine_kernel(og_ref, out_ref):
    o = og_ref[:, :, 0:D]
    l = jnp.max(og_ref[:, :, D:P], axis=2, keepdims=True)
    m = jnp.max(l, axis=0, keepdims=True)
    p = jnp.exp(l - m)
    s = jnp.sum(p, axis=0, keepdims=True)
    w = p / s
    out_ref[0] = jnp.sum(o * w, axis=0)


def _combine(og3):
    return pl.pallas_call(
        _combine_kernel,
        grid=(B, S // CCH),
        in_specs=[
            pl.BlockSpec((H, CCH, P), lambda b, c: (b, c, 0)),
        ],
        out_specs=pl.BlockSpec((1, CCH, D), lambda b, c: (b, c, 0)),
        out_shape=jax.ShapeDtypeStruct((B, S, D), jnp.float32),
    )(og3)


def kernel(qk, v, random_rotations):
    rotw = jnp.transpose(random_rotations[0], (1, 0, 2)).reshape(H, D, NB // 2)
    bkt4, fg4, qt04, ktl4 = _hash_sort(qk, rotw)
    buckets = bkt4.reshape(B, H * S)
    fg = fg4.reshape(B * H * S)

    qkv = jnp.concatenate([qk, v], axis=-1).reshape(B * S, P)
    sqkv = _sc_scatter(qkv, fg)

    kv_last = sqkv.reshape(B, H * S, P)[:, -BKT:, :]
    sog = _attention(sqkv, kv_last, qt04, ktl4)

    og = _sc_gather(sog, fg)
    out = _combine(og.reshape(B * H, S, P))
    return out, buckets


# speedup vs baseline: 6.2073x; 1.0823x over previous
import jax
import jax.numpy as jnp
from jax import lax
from jax.experimental import pallas as pl
from jax.experimental.pallas import tpu as pltpu
from jax.experimental.pallas import tpu_sc as plsc

B = 2
S = 4096
D = 64
H = 8
NB = 64
BKT = 64
CHUNK = 1024
NCHK = S // CHUNK
BINS_PER = 32
ROWS = BINS_PER * BKT
NCH = (H * S) // ROWS
CCH = 512
P = 2 * D

NW = 32
TPT = (B * H * S) // NW
K = 128
NCK = TPT // K

_SC_MESH = dict(core_axis_name="c", subcore_axis_name="s",
                num_cores=2, num_subcores=16)


def _hash_sort_kernel(qk_ref, rot_ref, bkt_ref, fg_ref, qt0_ref, ktl_ref):
    b = pl.program_id(0)
    h = pl.program_id(1)
    qk = qk_ref[0]
    rot = rot_ref[0]
    r = jnp.dot(qk, rot, preferred_element_type=jnp.float32)
    scores = jnp.concatenate([r, -r], axis=1)
    m = jnp.max(scores, axis=1, keepdims=True)
    lane = lax.broadcasted_iota(jnp.int32, (S, NB), 1)
    bkt = jnp.min(jnp.where(scores == m, lane, NB), axis=1, keepdims=True)
    onehot = (bkt == lane).astype(jnp.float32)
    bktf = bkt.astype(jnp.float32)
    counts = jnp.sum(onehot, axis=0, keepdims=True)
    i0 = lax.broadcasted_iota(jnp.int32, (NB, NB), 0)
    i1 = lax.broadcasted_iota(jnp.int32, (NB, NB), 1)
    upper = (i0 < i1).astype(jnp.float32)
    offs = jnp.dot(counts, upper, preferred_element_type=jnp.float32,
                   precision=lax.Precision.HIGHEST)
    c0 = lax.broadcasted_iota(jnp.int32, (CHUNK, CHUNK), 0)
    c1 = lax.broadcasted_iota(jnp.int32, (CHUNK, CHUNK), 1)
    lower = (c0 >= c1).astype(jnp.bfloat16)
    ident = (c0 == c1).astype(jnp.float32)
    flatbase = ((b * H + h) * S).astype(jnp.float32)
    hoff = (h * NB).astype(jnp.float32)
    l64 = lax.broadcasted_iota(jnp.int32, (CHUNK, BKT), 1).astype(jnp.float32)
    tcol = lax.broadcasted_iota(jnp.int32, (CHUNK, 1), 0).astype(jnp.float32)
    carry = jnp.zeros((1, NB), jnp.float32)
    qt0a = jnp.zeros((1, BKT), jnp.float32)
    ktla = jnp.zeros((1, BKT), jnp.float32)
    poscols = []
    for ci in range(NCHK):
        sl = slice(ci * CHUNK, (ci + 1) * CHUNK)
        seg = onehot[sl]
        incl = jnp.dot(lower, seg.astype(jnp.bfloat16),
                       preferred_element_type=jnp.float32)
        base = offs + carry - 1.0
        pos = jnp.sum(seg * (incl + base), axis=1, keepdims=True)
        poscols.append(pos)
        carry = carry + incl[CHUNK - 1:CHUNK, :]
        tvals = tcol + float(ci * CHUNK)
        oh0 = (pos == l64).astype(jnp.float32)
        ohl = (pos == l64 + float(S - BKT)).astype(jnp.float32)
        qt0a = qt0a + lax.dot_general(tvals, oh0, (((0,), (0,)), ((), ())),
                                      preferred_element_type=jnp.float32,
                                      precision=lax.Precision.HIGHEST)
        ktla = ktla + lax.dot_general(tvals, ohl, (((0,), (0,)), ((), ())),
                                      preferred_element_type=jnp.float32,
                                      precision=lax.Precision.HIGHEST)
    bktcols = [bktf[ci * CHUNK:(ci + 1) * CHUNK] for ci in range(NCHK)]
    colm = jnp.concatenate(poscols + bktcols, axis=1)
    rowm = lax.dot_general(colm, ident, (((0,), (0,)), ((), ())),
                           preferred_element_type=jnp.float32,
                           precision=lax.Precision.HIGHEST)
    fg_ref[0, 0] = (rowm[0:NCHK] + flatbase).astype(jnp.int32)
    bkt_ref[0, 0] = (rowm[NCHK:2 * NCHK] + hoff).astype(jnp.int32)
    qt0_ref[0, 0] = qt0a
    ktl_ref[0, 0] = ktla


def _hash_sort(qk, rotw):
    return pl.pallas_call(
        _hash_sort_kernel,
        grid=(B, H),
        in_specs=[
            pl.BlockSpec((1, S, D), lambda b, h: (b, 0, 0)),
            pl.BlockSpec((1, D, NB // 2), lambda b, h: (h, 0, 0)),
        ],
        out_specs=[
            pl.BlockSpec((1, 1, NCHK, CHUNK), lambda b, h: (b, h, 0, 0)),
            pl.BlockSpec((1, 1, NCHK, CHUNK), lambda b, h: (b, h, 0, 0)),
            pl.BlockSpec((1, 1, 1, BKT), lambda b, h: (b, h, 0, 0)),
            pl.BlockSpec((1, 1, 1, BKT), lambda b, h: (b, h, 0, 0)),
        ],
        out_shape=[
            jax.ShapeDtypeStruct((B, H, NCHK, CHUNK), jnp.int32),
            jax.ShapeDtypeStruct((B, H, NCHK, CHUNK), jnp.int32),
            jax.ShapeDtypeStruct((B, H, 1, BKT), jnp.float32),
            jax.ShapeDtypeStruct((B, H, 1, BKT), jnp.float32),
        ],
    )(qk, rotw)


def _sc_wid():
    return lax.axis_index("s") * 2 + lax.axis_index("c")


def _sc_scatter_body(qkv_hbm, fg_hbm, sqkv_hbm,
                     idx0, idx1, rows0, rows1, sem_in, sem_out):
    wid = _sc_wid()
    pair = wid // 2
    half = wid % 2
    b = pair // H
    fgbase = pair * S + half * (S // 2)
    qkbase = b * S + half * (S // 2)
    idx = (idx0, idx1)
    rows = (rows0, rows1)
    pend = {}
    for ck in range(NCK):
        s = ck % 2
        if ck >= 2:
            pend[s].wait()
        t0 = ck * K
        l1 = pltpu.async_copy(fg_hbm.at[pl.ds(fgbase + t0, K)], idx[s], sem_in)
        l2 = pltpu.async_copy(qkv_hbm.at[pl.ds(qkbase + t0, K)], rows[s], sem_in)
        l1.wait()
        l2.wait()
        pend[s] = pltpu.async_copy(rows[s], sqkv_hbm.at[idx[s]], sem_out)
    for s in (0, 1):
        pend[s].wait()


def _sc_scatter(qkv, fg):
    fn = pl.kernel(
        _sc_scatter_body,
        out_type=jax.ShapeDtypeStruct((B * H * S, P), jnp.float32),
        mesh=plsc.VectorSubcoreMesh(**_SC_MESH),
        compiler_params=pltpu.CompilerParams(use_tc_tiling_on_sc=False),
        scratch_types=[
            pltpu.VMEM((K,), jnp.int32),
            pltpu.VMEM((K,), jnp.int32),
            pltpu.VMEM((K, P), jnp.float32),
            pltpu.VMEM((K, P), jnp.float32),
            pltpu.SemaphoreType.DMA,
            pltpu.SemaphoreType.DMA,
        ],
    )
    return fn(qkv, fg)


def _sc_gather_body(sog_hbm, fg_hbm, og_hbm,
                    idx0, idx1, rows0, rows1, sem_in, sem_out):
    wid = _sc_wid()
    base = (wid // 2) * S + (wid % 2) * (S // 2)
    idx = (idx0, idx1)
    rows = (rows0, rows1)
    pend = {}
    for ck in range(NCK):
        s = ck % 2
        if ck >= 2:
            pend[s].wait()
        t0 = ck * K
        l1 = pltpu.async_copy(fg_hbm.at[pl.ds(base + t0, K)], idx[s], sem_in)
        l1.wait()
        g = pltpu.async_copy(sog_hbm.at[idx[s]], rows[s], sem_in)
        g.wait()
        pend[s] = pltpu.async_copy(rows[s], og_hbm.at[pl.ds(base + t0, K)],
                                   sem_out)
    for s in (0, 1):
        pend[s].wait()


def _sc_gather(sog, fg):
    fn = pl.kernel(
        _sc_gather_body,
        out_type=jax.ShapeDtypeStruct((B * H * S, P), jnp.float32),
        mesh=plsc.VectorSubcoreMesh(**_SC_MESH),
        compiler_params=pltpu.CompilerParams(use_tc_tiling_on_sc=False),
        scratch_types=[
            pltpu.VMEM((K,), jnp.int32),
            pltpu.VMEM((K,), jnp.int32),
            pltpu.VMEM((K, P), jnp.float32),
            pltpu.VMEM((K, P), jnp.float32),
            pltpu.SemaphoreType.DMA,
            pltpu.SemaphoreType.DMA,
        ],
    )
    return fn(sog, fg)


def _attn_kernel(kv_ref, kv_last_ref, qt_ref, ktl_ref, sog_ref, kvprev):
    c = pl.program_id(1)

    @pl.when(c == 0)
    def _():
        kvprev[...] = kv_last_ref[0]

    cur = kv_ref[...]
    prev = jnp.concatenate([kvprev[...], cur[:ROWS - BKT]], axis=0)
    kvprev[...] = cur[ROWS - BKT:ROWS]

    cur3 = cur.reshape(BINS_PER, BKT, P)
    prev3 = prev.reshape(BINS_PER, BKT, P)
    q = cur3[:, :, 0:D]
    kk = jnp.concatenate([cur3[:, :, 0:D], prev3[:, :, 0:D]], axis=1)
    vv = jnp.concatenate([cur3[:, :, D:P], prev3[:, :, D:P]], axis=1)
    norm = jnp.sqrt(jnp.sum(kk * kk, axis=2, keepdims=True))
    bk = kk / (norm + 1e-6)
    dots = lax.dot_general(q, bk, (((2,), (2,)), ((0,), (0,))),
                           preferred_element_type=jnp.float32)
    dots = dots * (D ** -0.5)

    i1 = lax.broadcasted_iota(jnp.int32, (BINS_PER, BKT, 2 * BKT), 1)
    i2 = lax.broadcasted_iota(jnp.int32, (BINS_PER, BKT, 2 * BKT), 2)
    mask = i1 == i2
    e0 = lax.broadcasted_iota(jnp.int32, (BKT, BKT), 0)
    e1 = lax.broadcasted_iota(jnp.int32, (BKT, BKT), 1)
    ident = (e0 == e1).astype(jnp.float32)
    qt = qt_ref[0, 0]
    ktl = ktl_ref[0, 0]
    qtT = lax.dot_general(ident, qt, (((1,), (1,)), ((), ())),
                          preferred_element_type=jnp.float32,
                          precision=lax.Precision.HIGHEST)
    ktlp = jnp.concatenate(
        [jnp.full((1, BKT), -1.0, jnp.float32), ktl], axis=1)
    lb = qtT == ktlp
    b0 = lax.broadcasted_iota(jnp.int32, (BINS_PER, BKT, 2 * BKT), 0) == 0
    mask = mask | (b0 & lb[None] & ((c % 2) == 0))
    dots = jnp.where(mask, -100000.0, dots)

    mx = jnp.max(dots, axis=2, keepdims=True)
    p = jnp.exp(dots - mx)
    sm = jnp.sum(p, axis=2, keepdims=True)
    lse = mx + jnp.log(sm)
    w = p / sm
    bo = lax.dot_general(w, vv, (((2,), (1,)), ((0,), (0,))),
                         preferred_element_type=jnp.float32)
    lse_b = jnp.broadcast_to(lse, (BINS_PER, BKT, D))
    sog_ref[...] = jnp.concatenate([bo, lse_b], axis=2).reshape(ROWS, P)


def _attention(sqkv, kv_last, qt0, ktl):
    return pl.pallas_call(
        _attn_kernel,
        grid=(B, NCH),
        in_specs=[
            pl.BlockSpec((ROWS, P), lambda b, c: (b * NCH + c, 0)),
            pl.BlockSpec((1, BKT, P), lambda b, c: (b, 0, 0)),
            pl.BlockSpec((1, 1, 1, BKT), lambda b, c: (b, c // 2, 0, 0)),
            pl.BlockSpec((1, 1, 1, BKT),
                         lambda b, c: (b, (c // 2 + H - 1) % H, 0, 0)),
        ],
        out_specs=pl.BlockSpec((ROWS, P), lambda b, c: (b * NCH + c, 0)),
        out_shape=jax.ShapeDtypeStruct((B * H * S, P), jnp.float32),
        scratch_shapes=[
            pltpu.VMEM((BKT, P), jnp.float32),
        ],
    )(sqkv, kv_last, qt0, ktl)


def _combine_kernel(og_ref, out_ref):
    o = og_ref[:, :, 0:D]
    l = jnp.max(og_ref[:, :, D:P], axis=2, keepdims=True)
    m = jnp.max(l, axis=0, keepdims=True)
    p = jnp.exp(l - m)
    s = jnp.sum(p, axis=0, keepdims=True)
    w = p / s
    out_ref[0] = jnp.sum(o * w, axis=0)


def _combine(og3):
    return pl.pallas_call(
        _combine_kernel,
        grid=(B, S // CCH),
        in_specs=[
            pl.BlockSpec((H, CCH, P), lambda b, c: (b, c, 0)),
        ],
        out_specs=pl.BlockSpec((1, CCH, D), lambda b, c: (b, c, 0)),
        out_shape=jax.ShapeDtypeStruct((B, S, D), jnp.float32),
    )(og3)


def kernel(qk, v, random_rotations):
    rotw = jnp.transpose(random_rotations[0], (1, 0, 2)).reshape(H, D, NB // 2)
    bkt4, fg4, qt04, ktl4 = _hash_sort(qk, rotw)
    buckets = bkt4.reshape(B, H * S)
    fg = fg4.reshape(B * H * S)

    qkv = jnp.concatenate([qk, v], axis=-1).reshape(B * S, P)
    sqkv = _sc_scatter(qkv, fg)

    kv_last = sqkv.reshape(B, H * S, P)[:, -BKT:, :]
    sog = _attention(sqkv, kv_last, qt04, ktl4)

    og = _sc_gather(sog, fg)
    out = _combine(og.reshape(B * H, S, P))
    return out, buckets
